# baseline (speedup 1.0000x reference)
import functools

import jax
import jax.numpy as jnp
from jax import lax
from jax.experimental import pallas as pl
from jax.experimental.pallas import tpu as pltpu
from jax.experimental.pallas import tpu_sc as plsc

EMBED = 64
BATCH = 16384

_NC, _NS = 2, 16
_NW = _NC * _NS
_BPW = BATCH // _NW


@functools.cache
def _make_gather():
    mesh = plsc.VectorSubcoreMesh(
        core_axis_name="c", subcore_axis_name="s", num_cores=_NC)

    @functools.partial(
        pl.kernel,
        mesh=mesh,
        compiler_params=pltpu.CompilerParams(use_tc_tiling_on_sc=False),
        out_type=[
            jax.ShapeDtypeStruct((BATCH, EMBED), jnp.float32),
            jax.ShapeDtypeStruct((BATCH, EMBED), jnp.float32),
        ],
        scratch_types=[
            pltpu.VMEM((_BPW,), jnp.int32),
            pltpu.VMEM((_BPW,), jnp.float32),
            pltpu.VMEM((_BPW,), jnp.int32),
            pltpu.VMEM((_BPW,), jnp.float32),
            pltpu.SemaphoreType.DMA,
            pltpu.SemaphoreType.DMA,
        ],
    )
    def gather_kernel(user_hbm, movie_hbm, utab_hbm, mtab_hbm, uout_hbm,
                      mout_hbm, uidx_v, urows_v, midx_v, mrows_v, usem, msem):
        wid = lax.axis_index("s") * _NC + lax.axis_index("c")
        base = wid * _BPW
        pltpu.sync_copy(user_hbm.at[pl.ds(base, _BPW)], uidx_v)
        pltpu.sync_copy(movie_hbm.at[pl.ds(base, _BPW)], midx_v)
        ucp = pltpu.async_copy(utab_hbm.at[0].at[uidx_v], urows_v, usem)
        mcp = pltpu.async_copy(mtab_hbm.at[0].at[midx_v], mrows_v, msem)
        ucp.wait()
        mcp.wait()

    return gather_kernel


_BLK = 2048


def _mlp_body(u_ref, m_ref, w1u_ref, w1m_ref, b1_ref, w2_ref, b2_ref, w3_ref,
              b3_ref, w4_ref, b4_ref, w5_ref, b5_ref, out_ref):
    x = u_ref[...] @ w1u_ref[...] + m_ref[...] @ w1m_ref[...] + b1_ref[...]
    x = jnp.maximum(x, 0.0)
    x = jnp.maximum(x @ w2_ref[...] + b2_ref[...], 0.0)
    x = jnp.maximum(x @ w3_ref[...] + b3_ref[...], 0.0)
    x = jnp.maximum(x @ w4_ref[...] + b4_ref[...], 0.0)
    out_ref[...] = x @ w5_ref[...] + b5_ref[...]


def _mlp(u, m, W1u, W1m, b1, W2, b2, W3, b3, W4, b4, W5, b5):
    grid = (BATCH // _BLK,)
    row_spec = pl.BlockSpec((_BLK, EMBED), lambda i: (i, 0))
    full = lambda a: pl.BlockSpec(a.shape, lambda i: (0,) * a.ndim)
    in_specs = [row_spec, row_spec] + [
        full(a) for a in (W1u, W1m, b1, W2, b2, W3, b3, W4, b4, W5, b5)
    ]
    return pl.pallas_call(
        _mlp_body,
        grid=grid,
        in_specs=in_specs,
        out_specs=pl.BlockSpec((_BLK, 1), lambda i: (i, 0)),
        out_shape=jax.ShapeDtypeStruct((BATCH, 1), jnp.float32),
        compiler_params=pltpu.CompilerParams(
            dimension_semantics=("parallel",),
        ),
    )(u, m, W1u, W1m, b1, W2, b2, W3, b3, W4, b4, W5, b5)


def kernel(user, movie, user_table, movie_table, W1, b1, W2, b2, W3, b3, W4,
           b4, W5, b5):
    u, m = _make_gather()(user.astype(jnp.int32), movie.astype(jnp.int32),
                          user_table.T, movie_table.T)
    return _mlp(u, m, W1[:EMBED], W1[EMBED:], b1.reshape(1, -1),
                W2, b2.reshape(1, -1), W3, b3.reshape(1, -1),
                W4, b4.reshape(1, -1), W5, b5.reshape(1, -1))


# speedup vs baseline: 8.8026x; 8.8026x over previous
import functools

import jax
import jax.numpy as jnp
from jax import lax
from jax.experimental import pallas as pl
from jax.experimental.pallas import tpu as pltpu
from jax.experimental.pallas import tpu_sc as plsc

EMBED = 64
BATCH = 16384

_NC, _NS = 2, 16
_NW = _NC * _NS
_BPW = BATCH // _NW


@functools.cache
def _make_gather():
    mesh = plsc.VectorSubcoreMesh(
        core_axis_name="c", subcore_axis_name="s", num_cores=_NC)

    @functools.partial(
        pl.kernel,
        mesh=mesh,
        compiler_params=pltpu.CompilerParams(use_tc_tiling_on_sc=False),
        out_type=[
            jax.ShapeDtypeStruct((BATCH, EMBED), jnp.float32),
            jax.ShapeDtypeStruct((BATCH, EMBED), jnp.float32),
        ],
        scratch_types=[
            pltpu.VMEM((_BPW,), jnp.int32),
            pltpu.VMEM((_BPW, EMBED), jnp.float32),
            pltpu.VMEM((_BPW,), jnp.int32),
            pltpu.VMEM((_BPW, EMBED), jnp.float32),
            pltpu.SemaphoreType.DMA,
            pltpu.SemaphoreType.DMA,
        ],
    )
    def gather_kernel(user_hbm, movie_hbm, utab_hbm, mtab_hbm, uout_hbm,
                      mout_hbm, uidx_v, urows_v, midx_v, mrows_v, usem, msem):
        wid = lax.axis_index("s") * _NC + lax.axis_index("c")
        base = wid * _BPW
        pltpu.sync_copy(user_hbm.at[pl.ds(base, _BPW)], uidx_v)
        pltpu.sync_copy(movie_hbm.at[pl.ds(base, _BPW)], midx_v)
        ucp = pltpu.async_copy(utab_hbm.at[uidx_v], urows_v, usem)
        mcp = pltpu.async_copy(mtab_hbm.at[midx_v], mrows_v, msem)
        ucp.wait()
        pltpu.sync_copy(urows_v, uout_hbm.at[pl.ds(base, _BPW)])
        mcp.wait()
        pltpu.sync_copy(mrows_v, mout_hbm.at[pl.ds(base, _BPW)])

    return gather_kernel


_BLK = 2048


def _mlp_body(u_ref, m_ref, w1u_ref, w1m_ref, b1_ref, w2_ref, b2_ref, w3_ref,
              b3_ref, w4_ref, b4_ref, w5_ref, b5_ref, out_ref):
    x = u_ref[...] @ w1u_ref[...] + m_ref[...] @ w1m_ref[...] + b1_ref[...]
    x = jnp.maximum(x, 0.0)
    x = jnp.maximum(x @ w2_ref[...] + b2_ref[...], 0.0)
    x = jnp.maximum(x @ w3_ref[...] + b3_ref[...], 0.0)
    x = jnp.maximum(x @ w4_ref[...] + b4_ref[...], 0.0)
    out_ref[...] = x @ w5_ref[...] + b5_ref[...]


def _mlp(u, m, W1u, W1m, b1, W2, b2, W3, b3, W4, b4, W5, b5):
    grid = (BATCH // _BLK,)
    row_spec = pl.BlockSpec((_BLK, EMBED), lambda i: (i, 0))
    full = lambda a: pl.BlockSpec(a.shape, lambda i: (0,) * a.ndim)
    in_specs = [row_spec, row_spec] + [
        full(a) for a in (W1u, W1m, b1, W2, b2, W3, b3, W4, b4, W5, b5)
    ]
    return pl.pallas_call(
        _mlp_body,
        grid=grid,
        in_specs=in_specs,
        out_specs=pl.BlockSpec((_BLK, 1), lambda i: (i, 0)),
        out_shape=jax.ShapeDtypeStruct((BATCH, 1), jnp.float32),
        compiler_params=pltpu.CompilerParams(
            dimension_semantics=("parallel",),
        ),
    )(u, m, W1u, W1m, b1, W2, b2, W3, b3, W4, b4, W5, b5)


def kernel(user, movie, user_table, movie_table, W1, b1, W2, b2, W3, b3, W4,
           b4, W5, b5):
    u, m = _make_gather()(user.astype(jnp.int32), movie.astype(jnp.int32),
                          user_table, movie_table)
    return _mlp(u, m, W1[:EMBED], W1[EMBED:], b1.reshape(1, -1),
                W2, b2.reshape(1, -1), W3, b3.reshape(1, -1),
                W4, b4.reshape(1, -1), W5, b5.reshape(1, -1))
